# baseline (speedup 1.0000x reference)
import jax
import jax.numpy as jnp
from jax import lax
from jax.experimental import pallas as pl
from jax.experimental.pallas import tpu as pltpu
from jax.experimental.pallas import tpu_sc as plsc

_ND = 8
_NT = 4
_DW = 128
_TW = 512
_TC = 128


def _sc_body(q_hbm, pos_hbm, out_hbm, st, pct, qb0, qb1,
             s_pos, s_in0, s_in1, s_out0, s_out1):
    bsz = q_hbm.shape[0]
    wid = lax.axis_index("s") * 2 + lax.axis_index("c")
    d0 = (wid % _ND) * _DW
    tbase = (wid // _ND) * _TW
    iota = lax.iota(jnp.int32, 16)

    for tc in range(_TW // _TC):
        pltpu.sync_copy(
            pos_hbm.at[pl.ds(tbase + tc * _TC, _TC), pl.ds(d0, _DW)], st)

        @plsc.parallel_loop(0, (_TC // 16) * (_DW // 16))
        def _tr(bi):
            ti = bi // (_DW // 16)
            di = bi % (_DW // 16)
            tvec = ti * 16 + iota
            for j in range(16):
                dvec = di * 16 + ((iota + j) & 15)
                v = plsc.load_gather(st, [tvec, dvec])
                plsc.store_scatter(pct, [dvec, tc * _TC + tvec], v)

    qbufs, s_ins, s_outs = (qb0, qb1), (s_in0, s_in1), (s_out0, s_out1)
    steps = [(b, tc) for b in range(bsz) for tc in range(_TW // _TC)]

    def q_slice(b, tc):
        return (b, pl.ds(d0, _DW), pl.ds(tbase + tc * _TC, _TC))

    in_cp = {0: pltpu.async_copy(
        q_hbm.at[q_slice(*steps[0])], qbufs[0], s_ins[0])}
    out_cp = {}

    for s, (b, tc) in enumerate(steps):
        cur = qbufs[s % 2]
        in_cp[s].wait()
        if s + 1 < len(steps):
            if s >= 1:
                out_cp[s - 1].wait()
            in_cp[s + 1] = pltpu.async_copy(
                q_hbm.at[q_slice(*steps[s + 1])],
                qbufs[(s + 1) % 2], s_ins[(s + 1) % 2])

        tq = tc * _TC

        @plsc.parallel_loop(0, _DW)
        def _add_row(d1):
            for tv in range(_TC // 16):
                v = pct[d1, pl.ds(tq + tv * 16, 16)]
                plsc.addupdate(cur.at[d1, pl.ds(tv * 16, 16)], v)

        out_cp[s] = pltpu.async_copy(
            cur, out_hbm.at[q_slice(b, tc)], s_outs[s % 2])

    out_cp[len(steps) - 2].wait()
    out_cp[len(steps) - 1].wait()


def kernel(q, pos_embed):
    bsz, d_model, q_frm = q.shape
    mesh = plsc.VectorSubcoreMesh(core_axis_name="c", subcore_axis_name="s")
    f = pl.kernel(
        _sc_body,
        mesh=mesh,
        out_type=jax.ShapeDtypeStruct((bsz, d_model, q_frm), q.dtype),
        scratch_types=[
            pltpu.VMEM((_TC, _DW), jnp.float32),
            pltpu.VMEM((_DW, _TW), jnp.float32),
            pltpu.VMEM((_DW, _TC), jnp.float32),
            pltpu.VMEM((_DW, _TC), jnp.float32),
            pltpu.SemaphoreType.DMA,
            pltpu.SemaphoreType.DMA,
            pltpu.SemaphoreType.DMA,
            pltpu.SemaphoreType.DMA,
            pltpu.SemaphoreType.DMA,
        ],
        compiler_params=pltpu.CompilerParams(needs_layout_passes=False),
    )
    return f(q, pos_embed)


# speedup vs baseline: 2.6380x; 2.6380x over previous
import jax
import jax.numpy as jnp
from jax.experimental import pallas as pl
from jax.experimental.pallas import tpu as pltpu

_TB = 256


def _body(q_ref, pos_ref, out_ref):
    out_ref[...] = q_ref[...] + jnp.swapaxes(pos_ref[...], 0, 1)[None]


def kernel(q, pos_embed):
    bsz, d_model, q_frm = q.shape
    grid = (q_frm // _TB,)
    return pl.pallas_call(
        _body,
        grid=grid,
        in_specs=[
            pl.BlockSpec((bsz, d_model, _TB), lambda t: (0, 0, t)),
            pl.BlockSpec((_TB, d_model), lambda t: (t, 0)),
        ],
        out_specs=pl.BlockSpec((bsz, d_model, _TB), lambda t: (0, 0, t)),
        out_shape=jax.ShapeDtypeStruct((bsz, d_model, q_frm), q.dtype),
        compiler_params=pltpu.CompilerParams(
            dimension_semantics=("arbitrary",),
        ),
    )(q, pos_embed)
